# baseline (speedup 1.0000x reference)
import functools

import jax
import jax.numpy as jnp
from jax import lax
from jax.experimental import pallas as pl
from jax.experimental.pallas import tpu as pltpu
from jax.experimental.pallas import tpu_sc as plsc

D = 128
NC = 2
NS = 16
NW = NC * NS
CHUNK = 80
GRP = 2
NBUF = 4
SLOT = GRP * CHUNK


def _embed_body(n_slots, table_hbm, idx_hbm, out_hbm, idx_v, rows_v, *sems):
    gsem = sems[:NBUF]
    ssem = sems[NBUF:]
    n_super = n_slots // NBUF
    c = lax.axis_index("c")
    s = lax.axis_index("s")
    wid = s * NC + c
    base = wid * (n_slots * SLOT)
    pltpu.sync_copy(idx_hbm.at[wid], idx_v)

    def gathers(p, m):
        return [
            pltpu.make_async_copy(
                table_hbm.at[idx_v.at[p * GRP + k]],
                rows_v.at[m, pl.ds(k * CHUNK, CHUNK)], gsem[m])
            for k in range(GRP)
        ]

    def store(p, m):
        return pltpu.make_async_copy(
            rows_v.at[m], out_hbm.at[pl.ds(base + p * SLOT, SLOT)], ssem[m])

    for m in range(NBUF):
        for g in gathers(m, m):
            g.start()

    def gen(gi, carry):
        for m in range(NBUF):
            p = gi * NBUF + m
            for g in gathers(p, m):
                g.wait()
            store(p, m).start()
        for m in range(NBUF):
            p = gi * NBUF + m
            store(p, m).wait()
            for g in gathers(p + NBUF, m):
                g.start()
        return carry

    lax.fori_loop(0, n_super - 1, gen, 0)

    for m in range(NBUF):
        p = (n_super - 1) * NBUF + m
        for g in gathers(p, m):
            g.wait()
        store(p, m).start()
    for m in range(NBUF):
        p = (n_super - 1) * NBUF + m
        store(p, m).wait()


def kernel(x, table):
    bsz, hist = x.shape
    n_total = bsz * hist
    assert n_total % (NW * SLOT) == 0
    n_slots = n_total // (NW * SLOT)
    n_chunks = n_slots * GRP
    assert n_slots % NBUF == 0
    idx = jnp.transpose(x).reshape(NW, n_chunks, CHUNK).astype(jnp.int32)

    run = pl.kernel(
        functools.partial(_embed_body, n_slots),
        out_type=jax.ShapeDtypeStruct((n_total, D), table.dtype),
        mesh=plsc.VectorSubcoreMesh(core_axis_name="c", subcore_axis_name="s"),
        scratch_types=[
            pltpu.VMEM((n_chunks, CHUNK), jnp.int32),
            pltpu.VMEM((NBUF, SLOT, D), jnp.float32),
        ] + [pltpu.SemaphoreType.DMA] * (2 * NBUF),
    )
    out = run(table, idx)
    return jnp.transpose(out.reshape(hist, bsz, D), (1, 0, 2))


# speedup vs baseline: 1.0118x; 1.0118x over previous
import functools

import jax
import jax.numpy as jnp
from jax import lax
from jax.experimental import pallas as pl
from jax.experimental.pallas import tpu as pltpu
from jax.experimental.pallas import tpu_sc as plsc

D = 128
NC = 2
NS = 16
NW = NC * NS
CHUNK = 64
NBUF = 10


def _embed_body(n_chunks, table_hbm, idx_hbm, out_hbm, idx_v, rows_v, *sems):
    gsem = sems[:NBUF]
    ssem = sems[NBUF:]
    n_super = n_chunks // NBUF
    c = lax.axis_index("c")
    s = lax.axis_index("s")
    wid = s * NC + c
    base = wid * (n_chunks * CHUNK)
    pltpu.sync_copy(idx_hbm.at[wid], idx_v)

    def gather(j, b):
        return pltpu.make_async_copy(
            table_hbm.at[idx_v.at[j]], rows_v.at[b], gsem[b])

    def store(j, b):
        return pltpu.make_async_copy(
            rows_v.at[b], out_hbm.at[pl.ds(base + j * CHUNK, CHUNK)], ssem[b])

    for b in range(NBUF):
        gather(b, b).start()

    def gen(g, carry):
        for b in range(NBUF):
            j = g * NBUF + b
            gather(j, b).wait()
            store(j, b).start()
        for b in range(NBUF):
            j = g * NBUF + b
            store(j, b).wait()
            gather(j + NBUF, b).start()
        return carry

    lax.fori_loop(0, n_super - 1, gen, 0)

    for b in range(NBUF):
        j = (n_super - 1) * NBUF + b
        gather(j, b).wait()
        store(j, b).start()
    for b in range(NBUF):
        j = (n_super - 1) * NBUF + b
        store(j, b).wait()


def kernel(x, table):
    bsz, hist = x.shape
    n_total = bsz * hist
    assert n_total % (NW * CHUNK) == 0
    n_chunks = n_total // (NW * CHUNK)
    assert n_chunks % NBUF == 0
    idx = jnp.transpose(x).reshape(NW, n_chunks, CHUNK).astype(jnp.int32)

    run = pl.kernel(
        functools.partial(_embed_body, n_chunks),
        out_type=jax.ShapeDtypeStruct((n_total, D), table.dtype),
        mesh=plsc.VectorSubcoreMesh(core_axis_name="c", subcore_axis_name="s"),
        scratch_types=[
            pltpu.VMEM((n_chunks, CHUNK), jnp.int32),
            pltpu.VMEM((NBUF, CHUNK, D), jnp.float32),
        ] + [pltpu.SemaphoreType.DMA] * (2 * NBUF),
    )
    out = run(table, idx)
    return jnp.transpose(out.reshape(hist, bsz, D), (1, 0, 2))
